# Initial kernel scaffold; baseline (speedup 1.0000x reference)
import jax
import jax.numpy as jnp
from jax.experimental import pallas as pl


def kernel(x, pe):
    raise NotImplementedError("write your pallas kernel here")


# speedup vs baseline: 1.5679x; 1.5679x over previous
import functools

import jax
import jax.numpy as jnp
from jax import lax
from jax.experimental import pallas as pl
from jax.experimental.pallas import tpu as pltpu
from jax.experimental.pallas import tpu_sc as plsc

_NC = 2
_NS = 16
_NW = _NC * _NS


def _pe_broadcast_body(B, S, D, rows_per_worker, chunk, pe_hbm, out_hbm,
                       buf, sem):
    wid = lax.axis_index("s") * _NC + lax.axis_index("c")
    base = wid * rows_per_worker
    n_chunks = rows_per_worker // chunk
    for c in range(n_chunks):
        row0 = base + c * chunk
        pltpu.sync_copy(pe_hbm.at[pl.ds(row0, chunk), :], buf)
        copies = [
            pltpu.async_copy(buf, out_hbm.at[b, pl.ds(row0, chunk), :], sem)
            for b in range(B)
        ]
        for cp in copies:
            cp.wait()


def kernel(x, pe):
    B, S, _ = x.shape
    D = pe.shape[1]
    rows_per_worker = S // _NW
    chunk = min(rows_per_worker, 64)

    mesh = plsc.VectorSubcoreMesh(
        core_axis_name="c", subcore_axis_name="s",
        num_cores=_NC, num_subcores=_NS,
    )
    k = pl.kernel(
        functools.partial(_pe_broadcast_body, B, S, D, rows_per_worker,
                          chunk),
        out_type=jax.ShapeDtypeStruct((B, S, D), pe.dtype),
        mesh=mesh,
        scratch_types=[
            pltpu.VMEM((chunk, D), pe.dtype),
            pltpu.SemaphoreType.DMA,
        ],
    )
    return k(pe)
